# baseline (speedup 1.0000x reference)
import functools

import jax
import jax.numpy as jnp
from jax import lax
from jax.experimental import pallas as pl
from jax.experimental.pallas import tpu as pltpu

N = 2048
D = 1024
F = 2048
E = 8
BT = 256
G = (N * 2) // BT + E - 1
P = G * BT
NF = 4
FH = F // NF
NEG = -1e30


def _meta_body(x_ref, rw_ref, rb_ref, pos0_ref, pos1_ref, pw_ref, be_ref):
    x = x_ref[...]
    logits = jax.lax.dot_general(
        x, rw_ref[...], (((1,), (0,)), ((), ())),
        preferred_element_type=jnp.float32) + rb_ref[...]
    eio = jax.lax.broadcasted_iota(jnp.int32, (N, E), 1)
    m0 = jnp.max(logits, axis=1, keepdims=True)
    e0 = jnp.min(jnp.where(logits == m0, eio, E), axis=1, keepdims=True)
    l2 = jnp.where(eio == e0, NEG, logits)
    m1 = jnp.max(l2, axis=1, keepdims=True)
    e1 = jnp.min(jnp.where(l2 == m1, eio, E), axis=1, keepdims=True)
    w0 = 1.0 / (1.0 + jnp.exp(m1 - m0))
    w1 = 1.0 - w0

    oh0 = (eio == e0).astype(jnp.float32)
    oh1 = (eio == e1).astype(jnp.float32)
    s = oh0 + oh1

    rio = jax.lax.broadcasted_iota(jnp.int32, (N, N), 0)
    cio = jax.lax.broadcasted_iota(jnp.int32, (N, N), 1)
    tri = (rio > cio).astype(jnp.bfloat16)
    cum = jax.lax.dot_general(
        tri, s.astype(jnp.bfloat16), (((1,), (0,)), ((), ())),
        preferred_element_type=jnp.float32)

    counts = cum[N - 1:N, :] + s[N - 1:N, :]
    counts_i = counts.astype(jnp.int32)
    pc = (((counts_i + BT - 1) // BT) * BT).astype(jnp.float32)
    er = jax.lax.broadcasted_iota(jnp.int32, (E, E), 0)
    ec = jax.lax.broadcasted_iota(jnp.int32, (E, E), 1)
    mlt = (er < ec).astype(jnp.float32)
    po = jax.lax.dot_general(
        pc, mlt, (((1,), (0,)), ((), ())),
        preferred_element_type=jnp.float32)

    rank0 = jnp.sum(oh0 * cum, axis=1, keepdims=True)
    rank1 = jnp.sum(oh1 * cum, axis=1, keepdims=True)
    off0 = jnp.sum(oh0 * po, axis=1, keepdims=True)
    off1 = jnp.sum(oh1 * po, axis=1, keepdims=True)
    p0f = off0 + rank0
    p1f = off1 + rank1
    pos0_ref[...] = p0f.astype(jnp.int32)
    pos1_ref[...] = p1f.astype(jnp.int32)
    pw_ref[...] = jnp.concatenate([p0f, p1f, w0, w1], axis=1)

    end = po + pc
    gio = jax.lax.broadcasted_iota(jnp.int32, (32, 1), 0)
    owned = (gio.astype(jnp.float32) * BT >= end)
    be = jnp.sum(owned.astype(jnp.int32), axis=1, keepdims=True)
    be_ref[...] = jnp.minimum(be, E - 1)


def _moe_body(be_ref, p0r_ref, p1r_ref, pw_ref, x_ref,
              wg_ref, wu_ref, wd_ref, out_ref, rows_scr, ysum_scr):
    f = pl.program_id(0)
    g = pl.program_id(1)
    base = g * BT
    rs = pl.ds(base, BT)

    @pl.when((f == 0) & (g == 0))
    def _():
        out_ref[...] = jnp.zeros_like(out_ref)

    @pl.when(f == 0)
    def _():
        pio_c = jax.lax.broadcasted_iota(jnp.int32, (BT, 1), 0) + base
        a0 = (p0r_ref[...] == pio_c)
        a1 = (p1r_ref[...] == pio_c)
        gath = a0.astype(jnp.bfloat16) + a1.astype(jnp.bfloat16)
        rows = jax.lax.dot_general(
            gath, x_ref[...].astype(jnp.bfloat16), (((1,), (0,)), ((), ())),
            preferred_element_type=jnp.float32)
        rows_scr[rs, :] = rows.astype(jnp.bfloat16)

    rows = rows_scr[rs, :]
    gate = jax.lax.dot_general(
        rows, wg_ref[0].astype(jnp.bfloat16), (((1,), (0,)), ((), ())),
        preferred_element_type=jnp.float32)
    up = jax.lax.dot_general(
        rows, wu_ref[0].astype(jnp.bfloat16), (((1,), (0,)), ((), ())),
        preferred_element_type=jnp.float32)
    h = (gate * (1.0 / (1.0 + jnp.exp(-gate))) * up).astype(jnp.bfloat16)
    ypart = jax.lax.dot_general(
        h, wd_ref[0].astype(jnp.bfloat16), (((1,), (0,)), ((), ())),
        preferred_element_type=jnp.float32)

    @pl.when(f == 0)
    def _():
        ysum_scr[rs, :] = ypart.astype(jnp.bfloat16)

    @pl.when(f > 0)
    def _():
        acc = ysum_scr[rs, :].astype(jnp.float32) + ypart
        ysum_scr[rs, :] = acc.astype(jnp.bfloat16)

    @pl.when(f == NF - 1)
    def _():
        y = ysum_scr[rs, :]
        pio_r = (jax.lax.broadcasted_iota(jnp.int32, (1, BT), 1)
                 + base).astype(jnp.float32)
        a0t = (pw_ref[:, 0:1] == pio_r)
        a1t = (pw_ref[:, 1:2] == pio_r)
        wmat = (jnp.where(a0t, pw_ref[:, 2:3], 0.0)
                + jnp.where(a1t, pw_ref[:, 3:4], 0.0)).astype(jnp.bfloat16)
        for n2 in range(4):
            sl = slice(n2 * (N // 4), (n2 + 1) * (N // 4))
            out_ref[sl, :] += jax.lax.dot_general(
                wmat[sl, :], y, (((1,), (0,)), ((), ())),
                preferred_element_type=jnp.float32)


def kernel(x, router_w, router_b, w_gate, w_up, w_down):
    pos0, pos1, pw, be = pl.pallas_call(
        _meta_body,
        out_shape=[
            jax.ShapeDtypeStruct((N, 1), jnp.int32),
            jax.ShapeDtypeStruct((N, 1), jnp.int32),
            jax.ShapeDtypeStruct((N, 4), jnp.float32),
            jax.ShapeDtypeStruct((32, 1), jnp.int32),
        ],
        compiler_params=pltpu.CompilerParams(
            vmem_limit_bytes=128 * 1024 * 1024),
    )(x, router_w, router_b.reshape(1, E))

    be_flat = be.reshape(-1)[:G]
    pos0_r = pos0.reshape(1, N)
    pos1_r = pos1.reshape(1, N)

    moe_spec = pltpu.PrefetchScalarGridSpec(
        num_scalar_prefetch=1,
        grid=(NF, G),
        in_specs=[
            pl.BlockSpec((1, N), lambda f, g, be: (0, 0)),
            pl.BlockSpec((1, N), lambda f, g, be: (0, 0)),
            pl.BlockSpec((N, 4), lambda f, g, be: (0, 0)),
            pl.BlockSpec((N, D), lambda f, g, be: (0, 0)),
            pl.BlockSpec((1, D, FH), lambda f, g, be: (be[g], 0, f)),
            pl.BlockSpec((1, D, FH), lambda f, g, be: (be[g], 0, f)),
            pl.BlockSpec((1, FH, D), lambda f, g, be: (be[g], f, 0)),
        ],
        out_specs=pl.BlockSpec((N, D), lambda f, g, be: (0, 0)),
        scratch_shapes=[
            pltpu.VMEM((P, D), jnp.bfloat16),
            pltpu.VMEM((P, D), jnp.bfloat16),
        ],
    )
    out = pl.pallas_call(
        _moe_body,
        grid_spec=moe_spec,
        out_shape=jax.ShapeDtypeStruct((N, D), jnp.float32),
        compiler_params=pltpu.CompilerParams(
            dimension_semantics=("arbitrary", "arbitrary"),
            vmem_limit_bytes=128 * 1024 * 1024),
    )(be_flat, pos0_r, pos1_r, pw, x, w_gate, w_up, w_down)
    return out


# speedup vs baseline: 1.0383x; 1.0383x over previous
import functools

import jax
import jax.numpy as jnp
from jax import lax
from jax.experimental import pallas as pl
from jax.experimental.pallas import tpu as pltpu
from jax.experimental.pallas import tpu_sc as plsc

N = 2048
D = 1024
F = 2048
E = 8
BT = 256
G = (N * 2) // BT + E - 1
P = G * BT
NW = 32
TW = N // NW
NEG = -1e30


def _meta_body(x_ref, rw_ref, rb_ref,
               pos0_ref, pos1_ref, w0_ref, w1_ref, be_ref, nv_ref):
    x = x_ref[...]
    logits = jax.lax.dot_general(
        x, rw_ref[...], (((1,), (0,)), ((), ())),
        preferred_element_type=jnp.float32) + rb_ref[...]
    eio = jax.lax.broadcasted_iota(jnp.int32, (N, E), 1)
    m0 = jnp.max(logits, axis=1, keepdims=True)
    e0 = jnp.min(jnp.where(logits == m0, eio, E), axis=1, keepdims=True)
    l2 = jnp.where(eio == e0, NEG, logits)
    m1 = jnp.max(l2, axis=1, keepdims=True)
    e1 = jnp.min(jnp.where(l2 == m1, eio, E), axis=1, keepdims=True)
    w0 = 1.0 / (1.0 + jnp.exp(m1 - m0))
    w1 = 1.0 - w0

    oh0 = (eio == e0).astype(jnp.float32)
    oh1 = (eio == e1).astype(jnp.float32)
    s = oh0 + oh1

    rio = jax.lax.broadcasted_iota(jnp.int32, (N, N), 0)
    cio = jax.lax.broadcasted_iota(jnp.int32, (N, N), 1)
    tri = (rio > cio).astype(jnp.bfloat16)
    cum = jax.lax.dot_general(
        tri, s.astype(jnp.bfloat16), (((1,), (0,)), ((), ())),
        preferred_element_type=jnp.float32)

    counts = cum[N - 1:N, :] + s[N - 1:N, :]
    counts_i = counts.astype(jnp.int32)
    pc = (((counts_i + BT - 1) // BT) * BT).astype(jnp.float32)
    er = jax.lax.broadcasted_iota(jnp.int32, (E, E), 0)
    ec = jax.lax.broadcasted_iota(jnp.int32, (E, E), 1)
    mlt = (er < ec).astype(jnp.float32)
    po = jax.lax.dot_general(
        pc, mlt, (((1,), (0,)), ((), ())),
        preferred_element_type=jnp.float32)

    rank0 = jnp.sum(oh0 * cum, axis=1, keepdims=True)
    rank1 = jnp.sum(oh1 * cum, axis=1, keepdims=True)
    off0 = jnp.sum(oh0 * po, axis=1, keepdims=True)
    off1 = jnp.sum(oh1 * po, axis=1, keepdims=True)
    pos0_ref[...] = (off0 + rank0).astype(jnp.int32)
    pos1_ref[...] = (off1 + rank1).astype(jnp.int32)
    w0_ref[...] = w0
    w1_ref[...] = w1

    end = po + pc
    gio = jax.lax.broadcasted_iota(jnp.int32, (32, 1), 0)
    owned = (gio.astype(jnp.float32) * BT >= end)
    be = jnp.sum(owned.astype(jnp.int32), axis=1, keepdims=True)
    be = jnp.minimum(be, E - 1)
    be_ref[...] = be

    bio = jax.lax.broadcasted_iota(jnp.int32, (32, E), 1)
    bh = (be == bio)
    po_b = jnp.sum(jnp.where(bh, po, 0.0), axis=1, keepdims=True)
    c_b = jnp.sum(jnp.where(bh, counts, 0.0), axis=1, keepdims=True)
    start = gio.astype(jnp.float32) * BT - po_b
    nv = jnp.clip(c_b - start, 0.0, float(BT))
    nv_ref[...] = nv.astype(jnp.int32)


def _sc_scatter_body(x_hbm, p0_hbm, p1_hbm, xs_hbm, xv, i0v, i1v, sem0, sem1):
    wid = lax.axis_index("s") * 2 + lax.axis_index("c")
    base = wid * TW
    pltpu.sync_copy(x_hbm.at[pl.ds(base, TW)], xv)
    pltpu.sync_copy(p0_hbm.at[pl.ds(base, TW)], i0v)
    pltpu.sync_copy(p1_hbm.at[pl.ds(base, TW)], i1v)
    c0 = pltpu.async_copy(xv, xs_hbm.at[i0v], sem0)
    c1 = pltpu.async_copy(xv, xs_hbm.at[i1v], sem1)
    c0.wait()
    c1.wait()


_sc_scatter = functools.partial(
    pl.kernel,
    out_type=jax.ShapeDtypeStruct((P, D), jnp.float32),
    mesh=plsc.VectorSubcoreMesh(
        core_axis_name="c", subcore_axis_name="s",
        num_cores=2, num_subcores=16),
    scratch_types=[
        pltpu.VMEM((TW, D), jnp.float32),
        pltpu.VMEM((TW,), jnp.int32),
        pltpu.VMEM((TW,), jnp.int32),
        pltpu.SemaphoreType.DMA,
        pltpu.SemaphoreType.DMA,
    ],
)(_sc_scatter_body)


def _upgate_body(be_ref, nv_ref, xs_ref, wg_ref, wu_ref, h_ref):
    g = pl.program_id(0)
    nv = nv_ref[g]
    rio = jax.lax.broadcasted_iota(jnp.int32, (BT, 1), 0)
    rows = jnp.where(rio < nv, xs_ref[...], 0.0).astype(jnp.bfloat16)

    gate = jax.lax.dot_general(
        rows, wg_ref[0].astype(jnp.bfloat16), (((1,), (0,)), ((), ())),
        preferred_element_type=jnp.float32)
    up = jax.lax.dot_general(
        rows, wu_ref[0].astype(jnp.bfloat16), (((1,), (0,)), ((), ())),
        preferred_element_type=jnp.float32)
    h = gate * (1.0 / (1.0 + jnp.exp(-gate))) * up
    h_ref[...] = h.astype(jnp.bfloat16)


def _down_body(be_ref, h_ref, p0r_ref, p1r_ref, w0r_ref, w1r_ref,
               wd_ref, ys_ref):
    g = pl.program_id(0)
    base = g * BT

    y = jax.lax.dot_general(
        h_ref[...], wd_ref[0].astype(jnp.bfloat16), (((1,), (0,)), ((), ())),
        preferred_element_type=jnp.float32)

    pio_c = jax.lax.broadcasted_iota(jnp.int32, (BT, 1), 0) + base
    a0 = (p0r_ref[...] == pio_c)
    a1 = (p1r_ref[...] == pio_c)
    wmat = (jnp.where(a0, w0r_ref[...], 0.0)
            + jnp.where(a1, w1r_ref[...], 0.0)).astype(jnp.bfloat16)
    ones = jnp.ones((N, 1), dtype=jnp.bfloat16)
    wrow = jax.lax.dot_general(
        wmat, ones, (((1,), (0,)), ((), ())),
        preferred_element_type=jnp.float32)
    ys_ref[...] = y * wrow


def _sc_combine_body(ys_hbm, p0_hbm, p1_hbm, out_hbm,
                     r0v, r1v, i0v, i1v, sem0, sem1):
    wid = lax.axis_index("s") * 2 + lax.axis_index("c")
    base = wid * TW
    hc = TW // 2
    for c in range(2):
        cb = base + c * hc
        pltpu.sync_copy(p0_hbm.at[pl.ds(cb, hc)], i0v)
        pltpu.sync_copy(p1_hbm.at[pl.ds(cb, hc)], i1v)
        cp0 = pltpu.async_copy(ys_hbm.at[i0v], r0v, sem0)
        cp1 = pltpu.async_copy(ys_hbm.at[i1v], r1v, sem1)
        cp0.wait()
        cp1.wait()
        for t in range(hc):
            def _add(j, carry, t=t):
                sl = pl.ds(j * 16, 16)
                r0v[t, sl] = r0v[t, sl] + r1v[t, sl]
                return carry
            lax.fori_loop(0, D // 16, _add, 0)
        pltpu.sync_copy(r0v, out_hbm.at[pl.ds(cb, hc)])


_sc_combine = functools.partial(
    pl.kernel,
    out_type=jax.ShapeDtypeStruct((N, D), jnp.float32),
    mesh=plsc.VectorSubcoreMesh(
        core_axis_name="c", subcore_axis_name="s",
        num_cores=2, num_subcores=16),
    scratch_types=[
        pltpu.VMEM((TW // 2, D), jnp.float32),
        pltpu.VMEM((TW // 2, D), jnp.float32),
        pltpu.VMEM((TW // 2,), jnp.int32),
        pltpu.VMEM((TW // 2,), jnp.int32),
        pltpu.SemaphoreType.DMA,
        pltpu.SemaphoreType.DMA,
    ],
)(_sc_combine_body)


def kernel(x, router_w, router_b, w_gate, w_up, w_down):
    pos0, pos1, w0, w1, be, nv = pl.pallas_call(
        _meta_body,
        out_shape=[
            jax.ShapeDtypeStruct((N, 1), jnp.int32),
            jax.ShapeDtypeStruct((N, 1), jnp.int32),
            jax.ShapeDtypeStruct((N, 1), jnp.float32),
            jax.ShapeDtypeStruct((N, 1), jnp.float32),
            jax.ShapeDtypeStruct((32, 1), jnp.int32),
            jax.ShapeDtypeStruct((32, 1), jnp.int32),
        ],
        compiler_params=pltpu.CompilerParams(
            vmem_limit_bytes=128 * 1024 * 1024),
    )(x, router_w, router_b.reshape(1, E))

    be_flat = be.reshape(-1)[:G]
    nv_flat = nv.reshape(-1)[:G]
    pos0_r = pos0.reshape(1, N)
    pos1_r = pos1.reshape(1, N)

    x_sorted = _sc_scatter(x, pos0.reshape(-1), pos1.reshape(-1))

    upgate_spec = pltpu.PrefetchScalarGridSpec(
        num_scalar_prefetch=2,
        grid=(G,),
        in_specs=[
            pl.BlockSpec((BT, D), lambda g, be, nv: (g, 0)),
            pl.BlockSpec((1, D, F), lambda g, be, nv: (be[g], 0, 0)),
            pl.BlockSpec((1, D, F), lambda g, be, nv: (be[g], 0, 0)),
        ],
        out_specs=pl.BlockSpec((BT, F), lambda g, be, nv: (g, 0)),
    )
    hidden = pl.pallas_call(
        _upgate_body,
        grid_spec=upgate_spec,
        out_shape=jax.ShapeDtypeStruct((P, F), jnp.bfloat16),
        compiler_params=pltpu.CompilerParams(
            dimension_semantics=("arbitrary",),
            vmem_limit_bytes=128 * 1024 * 1024),
    )(be_flat, nv_flat, x_sorted, w_gate, w_up)

    down_spec = pltpu.PrefetchScalarGridSpec(
        num_scalar_prefetch=1,
        grid=(G,),
        in_specs=[
            pl.BlockSpec((BT, F), lambda g, be: (g, 0)),
            pl.BlockSpec((1, N), lambda g, be: (0, 0)),
            pl.BlockSpec((1, N), lambda g, be: (0, 0)),
            pl.BlockSpec((1, N), lambda g, be: (0, 0)),
            pl.BlockSpec((1, N), lambda g, be: (0, 0)),
            pl.BlockSpec((1, F, D), lambda g, be: (be[g], 0, 0)),
        ],
        out_specs=pl.BlockSpec((BT, D), lambda g, be: (g, 0)),
    )
    y_sorted = pl.pallas_call(
        _down_body,
        grid_spec=down_spec,
        out_shape=jax.ShapeDtypeStruct((P, D), jnp.float32),
        compiler_params=pltpu.CompilerParams(
            dimension_semantics=("arbitrary",),
            vmem_limit_bytes=128 * 1024 * 1024),
    )(be_flat, hidden, pos0_r, pos1_r,
      w0.reshape(1, N), w1.reshape(1, N), w_down)

    out = _sc_combine(y_sorted, pos0.reshape(-1), pos1.reshape(-1))
    return out


# speedup vs baseline: 1.2198x; 1.1748x over previous
import functools

import jax
import jax.numpy as jnp
from jax.experimental import pallas as pl
from jax.experimental.pallas import tpu as pltpu

N = 2048
D = 1024
F = 2048
E = 8
BT = 256
G = (N * 2) // BT + E - 1
P = G * BT
FH = F // 2
NEG = -1e30


def _meta_body(x_ref, rw_ref, rb_ref,
               pos0_ref, pos1_ref, w0_ref, w1_ref, be_ref):
    x = x_ref[...]
    logits = jax.lax.dot_general(
        x, rw_ref[...], (((1,), (0,)), ((), ())),
        preferred_element_type=jnp.float32) + rb_ref[...]
    eio = jax.lax.broadcasted_iota(jnp.int32, (N, E), 1)
    m0 = jnp.max(logits, axis=1, keepdims=True)
    e0 = jnp.min(jnp.where(logits == m0, eio, E), axis=1, keepdims=True)
    l2 = jnp.where(eio == e0, NEG, logits)
    m1 = jnp.max(l2, axis=1, keepdims=True)
    e1 = jnp.min(jnp.where(l2 == m1, eio, E), axis=1, keepdims=True)
    w0 = 1.0 / (1.0 + jnp.exp(m1 - m0))
    w1 = 1.0 - w0

    oh0 = (eio == e0).astype(jnp.float32)
    oh1 = (eio == e1).astype(jnp.float32)
    s = oh0 + oh1

    rio = jax.lax.broadcasted_iota(jnp.int32, (N, N), 0)
    cio = jax.lax.broadcasted_iota(jnp.int32, (N, N), 1)
    tri = (rio > cio).astype(jnp.bfloat16)
    cum = jax.lax.dot_general(
        tri, s.astype(jnp.bfloat16), (((1,), (0,)), ((), ())),
        preferred_element_type=jnp.float32)

    counts = cum[N - 1:N, :] + s[N - 1:N, :]
    counts_i = counts.astype(jnp.int32)
    pc = (((counts_i + BT - 1) // BT) * BT).astype(jnp.float32)
    er = jax.lax.broadcasted_iota(jnp.int32, (E, E), 0)
    ec = jax.lax.broadcasted_iota(jnp.int32, (E, E), 1)
    mlt = (er < ec).astype(jnp.float32)
    po = jax.lax.dot_general(
        pc, mlt, (((1,), (0,)), ((), ())),
        preferred_element_type=jnp.float32)

    rank0 = jnp.sum(oh0 * cum, axis=1, keepdims=True)
    rank1 = jnp.sum(oh1 * cum, axis=1, keepdims=True)
    off0 = jnp.sum(oh0 * po, axis=1, keepdims=True)
    off1 = jnp.sum(oh1 * po, axis=1, keepdims=True)
    pos0_ref[...] = (off0 + rank0).astype(jnp.int32)
    pos1_ref[...] = (off1 + rank1).astype(jnp.int32)
    w0_ref[...] = w0
    w1_ref[...] = w1

    end = po + pc
    gio = jax.lax.broadcasted_iota(jnp.int32, (32, 1), 0)
    owned = (gio.astype(jnp.float32) * BT >= end)
    be = jnp.sum(owned.astype(jnp.int32), axis=1, keepdims=True)
    be_ref[...] = jnp.minimum(be, E - 1)


def _moe_body(be_ref, xb_ref, p0r_ref, p1r_ref,
              wg_ref, wu_ref, wd_ref, ys_ref):
    g = pl.program_id(0)
    base = g * BT

    pio_c = jax.lax.broadcasted_iota(jnp.int32, (BT, 1), 0) + base
    a0 = (p0r_ref[...] == pio_c)
    a1 = (p1r_ref[...] == pio_c)
    gath = a0.astype(jnp.bfloat16) + a1.astype(jnp.bfloat16)
    rows = jax.lax.dot_general(
        gath, xb_ref[...], (((1,), (0,)), ((), ())),
        preferred_element_type=jnp.float32).astype(jnp.bfloat16)

    y = jnp.zeros((BT, D), dtype=jnp.float32)
    for f in range(2):
        wg_h = wg_ref[0][:, f * FH:(f + 1) * FH].astype(jnp.bfloat16)
        wu_h = wu_ref[0][:, f * FH:(f + 1) * FH].astype(jnp.bfloat16)
        wd_h = wd_ref[0][f * FH:(f + 1) * FH, :].astype(jnp.bfloat16)
        gate = jax.lax.dot_general(
            rows, wg_h, (((1,), (0,)), ((), ())),
            preferred_element_type=jnp.float32)
        up = jax.lax.dot_general(
            rows, wu_h, (((1,), (0,)), ((), ())),
            preferred_element_type=jnp.float32)
        h = (gate * (1.0 / (1.0 + jnp.exp(-gate))) * up).astype(jnp.bfloat16)
        y = y + jax.lax.dot_general(
            h, wd_h, (((1,), (0,)), ((), ())),
            preferred_element_type=jnp.float32)

    ys_ref[...] = y.astype(jnp.bfloat16)


def _comb_body(ys_ref, p0c_ref, p1c_ref, w0c_ref, w1c_ref, out_ref):
    pio_r = jax.lax.broadcasted_iota(jnp.int32, (1, P), 1)
    a0 = (p0c_ref[...] == pio_r)
    a1 = (p1c_ref[...] == pio_r)
    w = (jnp.where(a0, w0c_ref[...], 0.0)
         + jnp.where(a1, w1c_ref[...], 0.0)).astype(jnp.bfloat16)
    out_ref[...] = jax.lax.dot_general(
        w, ys_ref[...], (((1,), (0,)), ((), ())),
        preferred_element_type=jnp.float32)


def kernel(x, router_w, router_b, w_gate, w_up, w_down):
    pos0, pos1, w0, w1, be = pl.pallas_call(
        _meta_body,
        out_shape=[
            jax.ShapeDtypeStruct((N, 1), jnp.int32),
            jax.ShapeDtypeStruct((N, 1), jnp.int32),
            jax.ShapeDtypeStruct((N, 1), jnp.float32),
            jax.ShapeDtypeStruct((N, 1), jnp.float32),
            jax.ShapeDtypeStruct((32, 1), jnp.int32),
        ],
        compiler_params=pltpu.CompilerParams(
            vmem_limit_bytes=128 * 1024 * 1024),
    )(x, router_w, router_b.reshape(1, E))

    be_flat = be.reshape(-1)[:G]
    pos0_r = pos0.reshape(1, N)
    pos1_r = pos1.reshape(1, N)
    x_bf = x.astype(jnp.bfloat16)

    moe_spec = pltpu.PrefetchScalarGridSpec(
        num_scalar_prefetch=1,
        grid=(G,),
        in_specs=[
            pl.BlockSpec((N, D), lambda g, be: (0, 0)),
            pl.BlockSpec((1, N), lambda g, be: (0, 0)),
            pl.BlockSpec((1, N), lambda g, be: (0, 0)),
            pl.BlockSpec((1, D, F), lambda g, be: (be[g], 0, 0)),
            pl.BlockSpec((1, D, F), lambda g, be: (be[g], 0, 0)),
            pl.BlockSpec((1, F, D), lambda g, be: (be[g], 0, 0)),
        ],
        out_specs=pl.BlockSpec((BT, D), lambda g, be: (g, 0)),
    )
    y_sorted = pl.pallas_call(
        _moe_body,
        grid_spec=moe_spec,
        out_shape=jax.ShapeDtypeStruct((P, D), jnp.bfloat16),
        compiler_params=pltpu.CompilerParams(
            dimension_semantics=("arbitrary",),
            vmem_limit_bytes=128 * 1024 * 1024),
    )(be_flat, x_bf, pos0_r, pos1_r, w_gate, w_up, w_down)

    out = pl.pallas_call(
        _comb_body,
        grid=(N // BT,),
        in_specs=[
            pl.BlockSpec((P, D), lambda t: (0, 0)),
            pl.BlockSpec((BT, 1), lambda t: (t, 0)),
            pl.BlockSpec((BT, 1), lambda t: (t, 0)),
            pl.BlockSpec((BT, 1), lambda t: (t, 0)),
            pl.BlockSpec((BT, 1), lambda t: (t, 0)),
        ],
        out_specs=pl.BlockSpec((BT, D), lambda t: (t, 0)),
        out_shape=jax.ShapeDtypeStruct((N, D), jnp.float32),
        compiler_params=pltpu.CompilerParams(
            dimension_semantics=("arbitrary",),
            vmem_limit_bytes=128 * 1024 * 1024),
    )(y_sorted, pos0, pos1, w0, w1)
    return out


# speedup vs baseline: 1.2504x; 1.0251x over previous
import functools

import jax
import jax.numpy as jnp
from jax.experimental import pallas as pl
from jax.experimental.pallas import tpu as pltpu

N = 2048
D = 1024
F = 2048
E = 8
BT = 256
G = (N * 2) // BT + E - 1
P = G * BT
FH = F // 2
NEG = -1e30


def _meta_body(x_ref, rw_ref, rb_ref,
               pos0_ref, pos1_ref, w0_ref, w1_ref, be_ref, xb_ref):
    x = x_ref[...]
    xb_ref[...] = x.astype(jnp.bfloat16)
    logits = jax.lax.dot_general(
        x, rw_ref[...], (((1,), (0,)), ((), ())),
        preferred_element_type=jnp.float32) + rb_ref[...]
    eio = jax.lax.broadcasted_iota(jnp.int32, (N, E), 1)
    m0 = jnp.max(logits, axis=1, keepdims=True)
    e0 = jnp.min(jnp.where(logits == m0, eio, E), axis=1, keepdims=True)
    l2 = jnp.where(eio == e0, NEG, logits)
    m1 = jnp.max(l2, axis=1, keepdims=True)
    e1 = jnp.min(jnp.where(l2 == m1, eio, E), axis=1, keepdims=True)
    w0 = 1.0 / (1.0 + jnp.exp(m1 - m0))
    w1 = 1.0 - w0

    oh0 = (eio == e0).astype(jnp.float32)
    oh1 = (eio == e1).astype(jnp.float32)
    s = oh0 + oh1

    rio = jax.lax.broadcasted_iota(jnp.int32, (N, N), 0)
    cio = jax.lax.broadcasted_iota(jnp.int32, (N, N), 1)
    tri = (rio > cio).astype(jnp.bfloat16)
    cum = jax.lax.dot_general(
        tri, s.astype(jnp.bfloat16), (((1,), (0,)), ((), ())),
        preferred_element_type=jnp.float32)

    counts = cum[N - 1:N, :] + s[N - 1:N, :]
    counts_i = counts.astype(jnp.int32)
    pc = (((counts_i + BT - 1) // BT) * BT).astype(jnp.float32)
    er = jax.lax.broadcasted_iota(jnp.int32, (E, E), 0)
    ec = jax.lax.broadcasted_iota(jnp.int32, (E, E), 1)
    mlt = (er < ec).astype(jnp.float32)
    po = jax.lax.dot_general(
        pc, mlt, (((1,), (0,)), ((), ())),
        preferred_element_type=jnp.float32)

    rank0 = jnp.sum(oh0 * cum, axis=1, keepdims=True)
    rank1 = jnp.sum(oh1 * cum, axis=1, keepdims=True)
    off0 = jnp.sum(oh0 * po, axis=1, keepdims=True)
    off1 = jnp.sum(oh1 * po, axis=1, keepdims=True)
    pos0_ref[...] = (off0 + rank0).astype(jnp.int32)
    pos1_ref[...] = (off1 + rank1).astype(jnp.int32)
    w0_ref[...] = w0
    w1_ref[...] = w1

    end = po + pc
    gio = jax.lax.broadcasted_iota(jnp.int32, (32, 1), 0)
    owned = (gio.astype(jnp.float32) * BT >= end)
    be = jnp.sum(owned.astype(jnp.int32), axis=1, keepdims=True)
    be_ref[...] = jnp.minimum(be, E - 1)


def _moe_body(be_ref, xb_ref, p0r_ref, p1r_ref,
              wg_ref, wu_ref, wd_ref, ys_ref):
    g = pl.program_id(0)
    base = g * BT

    pio_c = jax.lax.broadcasted_iota(jnp.int32, (BT, 1), 0) + base
    a0 = (p0r_ref[...] == pio_c)
    a1 = (p1r_ref[...] == pio_c)
    gath = a0.astype(jnp.bfloat16) + a1.astype(jnp.bfloat16)
    rows = jax.lax.dot_general(
        gath, xb_ref[...], (((1,), (0,)), ((), ())),
        preferred_element_type=jnp.float32).astype(jnp.bfloat16)

    y = jnp.zeros((BT, D), dtype=jnp.float32)
    for f in range(2):
        wg_h = wg_ref[0][:, f * FH:(f + 1) * FH].astype(jnp.bfloat16)
        wu_h = wu_ref[0][:, f * FH:(f + 1) * FH].astype(jnp.bfloat16)
        wd_h = wd_ref[0][f * FH:(f + 1) * FH, :].astype(jnp.bfloat16)
        gate = jax.lax.dot_general(
            rows, wg_h, (((1,), (0,)), ((), ())),
            preferred_element_type=jnp.float32)
        up = jax.lax.dot_general(
            rows, wu_h, (((1,), (0,)), ((), ())),
            preferred_element_type=jnp.float32)
        h = (gate * (1.0 / (1.0 + jnp.exp(-gate))) * up).astype(jnp.bfloat16)
        y = y + jax.lax.dot_general(
            h, wd_h, (((1,), (0,)), ((), ())),
            preferred_element_type=jnp.float32)

    ys_ref[...] = y.astype(jnp.bfloat16)


def _comb_body(ys_ref, p0c_ref, p1c_ref, w0c_ref, w1c_ref, out_ref):
    pio_r = jax.lax.broadcasted_iota(jnp.int32, (1, P), 1)
    a0 = (p0c_ref[...] == pio_r)
    a1 = (p1c_ref[...] == pio_r)
    w = (jnp.where(a0, w0c_ref[...], 0.0)
         + jnp.where(a1, w1c_ref[...], 0.0)).astype(jnp.bfloat16)
    out_ref[...] = jax.lax.dot_general(
        w, ys_ref[...], (((1,), (0,)), ((), ())),
        preferred_element_type=jnp.float32)


def kernel(x, router_w, router_b, w_gate, w_up, w_down):
    pos0, pos1, w0, w1, be, x_bf = pl.pallas_call(
        _meta_body,
        out_shape=[
            jax.ShapeDtypeStruct((N, 1), jnp.int32),
            jax.ShapeDtypeStruct((N, 1), jnp.int32),
            jax.ShapeDtypeStruct((N, 1), jnp.float32),
            jax.ShapeDtypeStruct((N, 1), jnp.float32),
            jax.ShapeDtypeStruct((32, 1), jnp.int32),
            jax.ShapeDtypeStruct((N, D), jnp.bfloat16),
        ],
        compiler_params=pltpu.CompilerParams(
            vmem_limit_bytes=128 * 1024 * 1024),
    )(x, router_w, router_b.reshape(1, E))

    be_flat = be.reshape(-1)[:G]
    pos0_r = pos0.reshape(1, N)
    pos1_r = pos1.reshape(1, N)

    moe_spec = pltpu.PrefetchScalarGridSpec(
        num_scalar_prefetch=1,
        grid=(G,),
        in_specs=[
            pl.BlockSpec((N, D), lambda g, be: (0, 0)),
            pl.BlockSpec((1, N), lambda g, be: (0, 0)),
            pl.BlockSpec((1, N), lambda g, be: (0, 0)),
            pl.BlockSpec((1, D, F), lambda g, be: (be[g], 0, 0)),
            pl.BlockSpec((1, D, F), lambda g, be: (be[g], 0, 0)),
            pl.BlockSpec((1, F, D), lambda g, be: (be[g], 0, 0)),
        ],
        out_specs=pl.BlockSpec((BT, D), lambda g, be: (g, 0)),
    )
    y_sorted = pl.pallas_call(
        _moe_body,
        grid_spec=moe_spec,
        out_shape=jax.ShapeDtypeStruct((P, D), jnp.bfloat16),
        compiler_params=pltpu.CompilerParams(
            dimension_semantics=("arbitrary",),
            vmem_limit_bytes=128 * 1024 * 1024),
    )(be_flat, x_bf, pos0_r, pos1_r, w_gate, w_up, w_down)

    out = pl.pallas_call(
        _comb_body,
        grid=(N // BT,),
        in_specs=[
            pl.BlockSpec((P, D), lambda t: (0, 0)),
            pl.BlockSpec((BT, 1), lambda t: (t, 0)),
            pl.BlockSpec((BT, 1), lambda t: (t, 0)),
            pl.BlockSpec((BT, 1), lambda t: (t, 0)),
            pl.BlockSpec((BT, 1), lambda t: (t, 0)),
        ],
        out_specs=pl.BlockSpec((BT, D), lambda t: (t, 0)),
        out_shape=jax.ShapeDtypeStruct((N, D), jnp.float32),
        compiler_params=pltpu.CompilerParams(
            dimension_semantics=("arbitrary",),
            vmem_limit_bytes=128 * 1024 * 1024),
    )(y_sorted, pos0, pos1, w0, w1)
    return out
